# baseline (speedup 1.0000x reference)
import functools

import jax
import jax.numpy as jnp
from jax import lax
from jax.experimental import pallas as pl
from jax.experimental.pallas import tpu as pltpu
from jax.experimental.pallas import tpu_sc as plsc

N = 10000
E = 320000
D = 128

NW = 32
EPW = E // NW
K = 80
CH = EPW // K
NPAD = 10240
RPT = NPAD // 16

BE = 640
BN = 2000

_mesh = plsc.VectorSubcoreMesh(core_axis_name="c", subcore_axis_name="s")


DW = D // 2
_MSK = -65536


def _gather_body(a_hbm, b_hbm, ic_hbm, g_hbm,
                 ic, sh_a, va, vb, sic, sga, sgb):
    c = lax.axis_index("c")
    s = lax.axis_index("s")
    wid = s * 2 + c
    for t in range(4):
        pltpu.async_copy(ic_hbm.at[wid, t], ic.at[t], sic[t])
    rr = pl.ds(s * RPT, RPT)
    pltpu.sync_copy(a_hbm.at[rr], sh_a.at[rr])
    plsc.subcore_barrier()

    def start_gather(j_slot, b):
        pltpu.async_copy(sh_a.at[ic.at[j_slot, 0]], va[b], sga[b])
        pltpu.async_copy(b_hbm.at[ic.at[j_slot, 1]], vb[b], sgb[b])

    pltpu.make_async_copy(ic_hbm.at[wid, 0], ic.at[0], sic[0]).wait()
    start_gather(0, 0)

    def do_chunk(j, t):
        b = t % 2

        @pl.when(j + 1 < CH)
        def _():
            pltpu.make_async_copy(ic_hbm.at[wid, 0], ic.at[(t + 1) % 4],
                                  sic[(t + 1) % 4]).wait()
            start_gather((t + 1) % 4, 1 - b)

        pltpu.make_async_copy(a_hbm.at[pl.ds(0, K)], va[b], sga[b]).wait()
        pltpu.make_async_copy(b_hbm.at[pl.ds(0, K)], vb[b], sgb[b]).wait()

        @pl.when(j + 4 < CH)
        def _():
            pltpu.async_copy(ic_hbm.at[wid, j + 4], ic.at[t], sic[t])

        def row(r, carry2):
            for cc in range(8):
                sl = pl.ds(cc * 16, 16)
                va[b][r, sl] = va[b][r, sl] + vb[b][r, sl]
            return carry2

        lax.fori_loop(0, K, row, 0, unroll=2)
        pltpu.sync_copy(va[b], g_hbm.at[pl.ds(wid * EPW + j * K, K)])

    def quad(j4, carry):
        for t in range(4):
            do_chunk(j4 * 4 + t, t)
        return carry

    lax.fori_loop(0, CH // 4, quad, 0)
    for j in range(CH - CH % 4, CH):
        do_chunk(j, j % 4)


@functools.partial(
    pl.kernel,
    out_type=jax.ShapeDtypeStruct((E, D), jnp.float32),
    mesh=_mesh,
    scratch_types=(
        [pltpu.VMEM((4, 2, K), jnp.int32)]
        + [pltpu.VMEM_SHARED((NPAD, D), jnp.float32)]
        + [pltpu.VMEM((K, D), jnp.float32)] * 4
        + [pltpu.SemaphoreType.DMA] * 8
    ),
)
def _sc_gather(a_hbm, b_hbm, ic_hbm, g_hbm, ic, sh_a, va0, va1, vb0, vb1,
               sic0, sic1, sic2, sic3, sga0, sga1, sgb0, sgb1):
    _gather_body(a_hbm, b_hbm, ic_hbm, g_hbm, ic, sh_a, (va0, va1),
                 (vb0, vb1), (sic0, sic1, sic2, sic3), (sga0, sga1),
                 (sgb0, sgb1))


@functools.partial(
    pl.kernel,
    out_type=jax.ShapeDtypeStruct((2, NPAD, D), jnp.float32),
    mesh=_mesh,
    scratch_types=[
        pltpu.VMEM((CH, K), jnp.int32),
        pltpu.VMEM((K, D), jnp.float32),
        pltpu.VMEM((K, D), jnp.float32),
        pltpu.VMEM_SHARED((NPAD, D), jnp.float32),
        pltpu.SemaphoreType.DMA,
        pltpu.SemaphoreType.DMA,
        pltpu.SemaphoreType.DMA,
        pltpu.SemaphoreType.DMA,
    ],
)
def _sc_scatter(enew_hbm, dst_hbm, out_hbm, idx, rows0, rows1, acc,
                sr0, sr1, sw0, sw1):
    c = lax.axis_index("c")
    s = lax.axis_index("s")
    wid = s * 2 + c
    rows = (rows0, rows1)
    sr = (sr0, sr1)
    sw = (sw0, sw1)

    def zrow(r, carry):
        for cc in range(8):
            rows0[r, pl.ds(cc * 16, 16)] = jnp.zeros((16,), jnp.float32)
        return carry

    lax.fori_loop(0, K, zrow, 0)

    def zacc(t, carry):
        pltpu.sync_copy(rows0, acc.at[pl.ds(s * RPT + t * K, K)])
        return carry

    lax.fori_loop(0, RPT // K, zacc, 0)
    plsc.subcore_barrier()

    pltpu.sync_copy(dst_hbm.at[wid], idx)
    pltpu.async_copy(enew_hbm.at[pl.ds(wid * EPW, K)], rows0, sr0)

    def do_chunk(j, b):
        @pl.when(j >= 1)
        def _():
            pltpu.make_async_copy(rows[1 - b], acc.at[idx.at[j]],
                                  sw[1 - b]).wait()

        @pl.when(j + 1 < CH)
        def _():
            pltpu.async_copy(enew_hbm.at[pl.ds(wid * EPW + (j + 1) * K, K)],
                             rows[1 - b], sr[1 - b])

        pltpu.make_async_copy(enew_hbm.at[pl.ds(wid * EPW + j * K, K)],
                              rows[b], sr[b]).wait()
        pltpu.async_copy(rows[b], acc.at[idx.at[j]], sw[b], add=True)

    def pair(j2, carry):
        do_chunk(j2 * 2, 0)
        do_chunk(j2 * 2 + 1, 1)
        return carry

    lax.fori_loop(0, CH // 2, pair, 0)
    if CH % 2:
        do_chunk(CH - 1, (CH - 1) % 2)
    pltpu.make_async_copy(rows[(CH - 1) % 2], acc.at[idx.at[CH - 1]],
                          sw[(CH - 1) % 2]).wait()
    plsc.subcore_barrier()

    pltpu.sync_copy(acc.at[pl.ds(s * RPT, RPT)], out_hbm.at[c].at[pl.ds(s * RPT, RPT)])


def _silu(v):
    return v * jax.nn.sigmoid(v)


def _bdot(u, w):
    return jnp.dot(u.astype(jnp.bfloat16), w.astype(jnp.bfloat16),
                   preferred_element_type=jnp.float32)


def _mlp_tail(h1, w2, b2, w3, b3, gamma, beta):
    h1 = _silu(h1)
    h2 = _silu(_bdot(h1, w2) + b2)
    v = _bdot(h2, w3) + b3
    mu = jnp.mean(v, axis=-1, keepdims=True)
    vc = v - mu
    var = jnp.mean(vc * vc, axis=-1, keepdims=True)
    return vc * lax.rsqrt(var + 1e-5) * gamma + beta


def _edge_kernel(g_ref, e_ref, w1e, b1, w2, b2, w3, b3, gamma, beta, out_ref):
    e = e_ref[...]
    h1 = g_ref[...] + _bdot(e, w1e[...]) + b1[...]
    out_ref[...] = _mlp_tail(h1, w2[...], b2[...], w3[...], b3[...],
                             gamma[...], beta[...]) + e


def _pre_kernel(x_ref, wd, ws, a_ref, b_ref):
    x = x_ref[...]
    a_ref[...] = _bdot(x, wd[...])
    b_ref[...] = _bdot(x, ws[...])


def _node_kernel(x_ref, o0_ref, o1_ref, v1x, v1o, b1, w2, b2, w3, b3,
                 gamma, beta, out_ref):
    x = x_ref[...]
    o = o0_ref[...] + o1_ref[...]
    h1 = (jnp.dot(x, v1x[...], preferred_element_type=jnp.float32)
          + jnp.dot(o, v1o[...], preferred_element_type=jnp.float32) + b1[...])
    out_ref[...] = _mlp_tail(h1, w2[...], b2[...], w3[...], b3[...],
                             gamma[...], beta[...]) + x


def _full(i):
    return (0, 0)


def _rows(i):
    return (i, 0)


_WSPEC = pl.BlockSpec((D, D), _full)
_VSPEC = pl.BlockSpec((1, D), _full)


def _edge_call(g, e, w1e, b1, w2, b2, w3, b3, gamma, beta):
    grid = (E // BE,)
    return pl.pallas_call(
        _edge_kernel,
        grid=grid,
        in_specs=[pl.BlockSpec((BE, D), _rows), pl.BlockSpec((BE, D), _rows),
                  _WSPEC, _VSPEC, _WSPEC, _VSPEC, _WSPEC, _VSPEC,
                  _VSPEC, _VSPEC],
        out_specs=pl.BlockSpec((BE, D), _rows),
        out_shape=jax.ShapeDtypeStruct((E, D), jnp.float32),
        compiler_params=pltpu.CompilerParams(
            dimension_semantics=("arbitrary",)),
    )(g, e, w1e, b1, w2, b2, w3, b3, gamma, beta)


def _pre_call(x, wd, ws):
    grid = (N // BN,)
    return pl.pallas_call(
        _pre_kernel,
        grid=grid,
        in_specs=[pl.BlockSpec((BN, D), _rows), _WSPEC, _WSPEC],
        out_specs=[pl.BlockSpec((BN, D), _rows), pl.BlockSpec((BN, D), _rows)],
        out_shape=[jax.ShapeDtypeStruct((N, D), jnp.float32),
                   jax.ShapeDtypeStruct((N, D), jnp.float32)],
        compiler_params=pltpu.CompilerParams(
            dimension_semantics=("arbitrary",)),
    )(x, wd, ws)


def _node_call(x, o0, o1, v1x, v1o, b1, w2, b2, w3, b3, gamma, beta):
    grid = (N // BN,)
    return pl.pallas_call(
        _node_kernel,
        grid=grid,
        in_specs=[pl.BlockSpec((BN, D), _rows), pl.BlockSpec((BN, D), _rows),
                  pl.BlockSpec((BN, D), _rows),
                  _WSPEC, _WSPEC, _VSPEC, _WSPEC, _VSPEC, _WSPEC, _VSPEC,
                  _VSPEC, _VSPEC],
        out_specs=pl.BlockSpec((BN, D), _rows),
        out_shape=jax.ShapeDtypeStruct((N, D), jnp.float32),
        compiler_params=pltpu.CompilerParams(
            dimension_semantics=("arbitrary",)),
    )(x, o0, o1, v1x, v1o, b1, w2, b2, w3, b3, gamma, beta)


def _row(v):
    return v.reshape(1, D)


def kernel(x, edge_index, edge_attr, params):
    dst3 = edge_index[1].reshape(NW, CH, K)
    src3 = edge_index[0].reshape(NW, CH, K)
    ic = jnp.stack([dst3, src3], axis=2)
    e = edge_attr
    for p in params:
        em = p["edge_mlp"]
        nm = p["node_mlp"]
        w1, b1 = em["l1"]
        w2, b2 = em["l2"]
        w3, b3 = em["l3"]
        gamma, beta = em["ln"]
        a, b = _pre_call(x, w1[:D], w1[D:2 * D])
        a_pad = jnp.pad(a, ((0, NPAD - N), (0, 0)))
        g = _sc_gather(a_pad, b, ic)
        e_new = _edge_call(g, e, w1[2 * D:], _row(b1), w2, _row(b2),
                           w3, _row(b3), _row(gamma), _row(beta))
        parts = _sc_scatter(e_new, dst3)
        o0 = parts[0, :N]
        o1 = parts[1, :N]
        v1, c1 = nm["l1"]
        v2, c2 = nm["l2"]
        v3, c3 = nm["l3"]
        ngamma, nbeta = nm["ln"]
        x = _node_call(x, o0, o1, v1[:D], v1[D:], _row(c1), v2, _row(c2),
                       v3, _row(c3), _row(ngamma), _row(nbeta))
        e = e_new
    return (x, e)


# speedup vs baseline: 1.0402x; 1.0402x over previous
import functools

import jax
import jax.numpy as jnp
from jax import lax
from jax.experimental import pallas as pl
from jax.experimental.pallas import tpu as pltpu
from jax.experimental.pallas import tpu_sc as plsc

N = 10000
E = 320000
D = 128

NW = 32
EPW = E // NW
K = 80
CH = EPW // K
NPAD = 10240
RPT = NPAD // 16

BE = 640
BN = 2000

_mesh = plsc.VectorSubcoreMesh(core_axis_name="c", subcore_axis_name="s")


DW = D // 2
_MSK = -65536


KG = 40
CHG = EPW // KG


def _gather_body(a_hbm, b_hbm, ic_hbm, g_hbm, ic, sh_a, va, vb,
                 sic, sga, sgb, ss):
    c = lax.axis_index("c")
    s = lax.axis_index("s")
    wid = s * 2 + c
    for t in range(4):
        pltpu.async_copy(ic_hbm.at[wid, t], ic.at[t], sic[t])
    rr = pl.ds(s * RPT, RPT)
    pltpu.sync_copy(a_hbm.at[rr], sh_a.at[rr])
    plsc.subcore_barrier()

    def start_gather(j_slot, b):
        pltpu.async_copy(sh_a.at[ic.at[j_slot, 0]], va[b], sga[b])
        pltpu.async_copy(b_hbm.at[ic.at[j_slot, 1]], vb[b], sgb[b])

    for t in range(2):
        pltpu.make_async_copy(ic_hbm.at[wid, 0], ic.at[t], sic[t]).wait()
        start_gather(t, t)

    def do_chunk(j, t):
        nb = (t + 2) % 4

        @pl.when(j >= 2)
        def _():
            pltpu.make_async_copy(va[nb], g_hbm.at[pl.ds(0, KG)],
                                  ss[nb]).wait()

        @pl.when(j + 2 < CHG)
        def _():
            pltpu.make_async_copy(ic_hbm.at[wid, 0], ic.at[nb],
                                  sic[nb]).wait()
            start_gather(nb, nb)

        pltpu.make_async_copy(a_hbm.at[pl.ds(0, KG)], va[t], sga[t]).wait()
        pltpu.make_async_copy(b_hbm.at[pl.ds(0, KG)], vb[t], sgb[t]).wait()

        @pl.when(j + 4 < CHG)
        def _():
            pltpu.async_copy(ic_hbm.at[wid, j + 4], ic.at[t], sic[t])

        def row(r, carry2):
            for cc in range(8):
                sl = pl.ds(cc * 16, 16)
                va[t][r, sl] = va[t][r, sl] + vb[t][r, sl]
            return carry2

        lax.fori_loop(0, KG, row, 0, unroll=2)
        pltpu.async_copy(va[t], g_hbm.at[pl.ds(wid * EPW + j * KG, KG)],
                         ss[t])

    def quad(j4, carry):
        for t in range(4):
            do_chunk(j4 * 4 + t, t)
        return carry

    lax.fori_loop(0, CHG // 4, quad, 0)
    for j in range(CHG - CHG % 4, CHG):
        do_chunk(j, j % 4)
    for j in range(CHG - 2, CHG):
        pltpu.make_async_copy(va[j % 4], g_hbm.at[pl.ds(0, KG)],
                              ss[j % 4]).wait()


@functools.partial(
    pl.kernel,
    out_type=jax.ShapeDtypeStruct((E, D), jnp.float32),
    mesh=_mesh,
    scratch_types=(
        [pltpu.VMEM((4, 2, KG), jnp.int32)]
        + [pltpu.VMEM_SHARED((NPAD, D), jnp.float32)]
        + [pltpu.VMEM((KG, D), jnp.float32)] * 8
        + [pltpu.SemaphoreType.DMA] * 16
    ),
)
def _sc_gather(a_hbm, b_hbm, ic_hbm, g_hbm, ic, sh_a, *rest):
    va = rest[0:4]
    vb = rest[4:8]
    sic = rest[8:12]
    sga = rest[12:16]
    sgb = rest[16:20]
    ss = rest[20:24]
    _gather_body(a_hbm, b_hbm, ic_hbm, g_hbm, ic, sh_a, va, vb,
                 sic, sga, sgb, ss)


@functools.partial(
    pl.kernel,
    out_type=jax.ShapeDtypeStruct((2, NPAD, D), jnp.float32),
    mesh=_mesh,
    scratch_types=[
        pltpu.VMEM((CH, K), jnp.int32),
        pltpu.VMEM((K, D), jnp.float32),
        pltpu.VMEM((K, D), jnp.float32),
        pltpu.VMEM_SHARED((NPAD, D), jnp.float32),
        pltpu.SemaphoreType.DMA,
        pltpu.SemaphoreType.DMA,
        pltpu.SemaphoreType.DMA,
        pltpu.SemaphoreType.DMA,
    ],
)
def _sc_scatter(enew_hbm, dst_hbm, out_hbm, idx, rows0, rows1, acc,
                sr0, sr1, sw0, sw1):
    c = lax.axis_index("c")
    s = lax.axis_index("s")
    wid = s * 2 + c
    rows = (rows0, rows1)
    sr = (sr0, sr1)
    sw = (sw0, sw1)

    def zrow(r, carry):
        for cc in range(8):
            rows0[r, pl.ds(cc * 16, 16)] = jnp.zeros((16,), jnp.float32)
        return carry

    lax.fori_loop(0, K, zrow, 0)

    def zacc(t, carry):
        pltpu.sync_copy(rows0, acc.at[pl.ds(s * RPT + t * K, K)])
        return carry

    lax.fori_loop(0, RPT // K, zacc, 0)
    plsc.subcore_barrier()

    pltpu.sync_copy(dst_hbm.at[wid], idx)
    pltpu.async_copy(enew_hbm.at[pl.ds(wid * EPW, K)], rows0, sr0)

    def do_chunk(j, b):
        @pl.when(j >= 1)
        def _():
            pltpu.make_async_copy(rows[1 - b], acc.at[idx.at[j]],
                                  sw[1 - b]).wait()

        @pl.when(j + 1 < CH)
        def _():
            pltpu.async_copy(enew_hbm.at[pl.ds(wid * EPW + (j + 1) * K, K)],
                             rows[1 - b], sr[1 - b])

        pltpu.make_async_copy(enew_hbm.at[pl.ds(wid * EPW + j * K, K)],
                              rows[b], sr[b]).wait()
        pltpu.async_copy(rows[b], acc.at[idx.at[j]], sw[b], add=True)

    def pair(j2, carry):
        do_chunk(j2 * 2, 0)
        do_chunk(j2 * 2 + 1, 1)
        return carry

    lax.fori_loop(0, CH // 2, pair, 0)
    if CH % 2:
        do_chunk(CH - 1, (CH - 1) % 2)
    pltpu.make_async_copy(rows[(CH - 1) % 2], acc.at[idx.at[CH - 1]],
                          sw[(CH - 1) % 2]).wait()
    plsc.subcore_barrier()

    pltpu.sync_copy(acc.at[pl.ds(s * RPT, RPT)], out_hbm.at[c].at[pl.ds(s * RPT, RPT)])


def _silu(v):
    return v * jax.nn.sigmoid(v)


def _bdot(u, w):
    return jnp.dot(u.astype(jnp.bfloat16), w.astype(jnp.bfloat16),
                   preferred_element_type=jnp.float32)


def _mlp_tail(h1, w2, b2, w3, b3, gamma, beta):
    h1 = _silu(h1)
    h2 = _silu(_bdot(h1, w2) + b2)
    v = _bdot(h2, w3) + b3
    mu = jnp.mean(v, axis=-1, keepdims=True)
    vc = v - mu
    var = jnp.mean(vc * vc, axis=-1, keepdims=True)
    return vc * lax.rsqrt(var + 1e-5) * gamma + beta


def _edge_kernel(g_ref, e_ref, w1e, b1, w2, b2, w3, b3, gamma, beta, out_ref):
    e = e_ref[...]
    h1 = g_ref[...] + _bdot(e, w1e[...]) + b1[...]
    out_ref[...] = _mlp_tail(h1, w2[...], b2[...], w3[...], b3[...],
                             gamma[...], beta[...]) + e


def _pre_kernel(x_ref, wd, ws, a_ref, b_ref):
    x = x_ref[...]
    a_ref[...] = _bdot(x, wd[...])
    b_ref[...] = _bdot(x, ws[...])


def _node_kernel(x_ref, o0_ref, o1_ref, v1x, v1o, b1, w2, b2, w3, b3,
                 gamma, beta, out_ref):
    x = x_ref[...]
    o = o0_ref[...] + o1_ref[...]
    h1 = (jnp.dot(x, v1x[...], preferred_element_type=jnp.float32)
          + jnp.dot(o, v1o[...], preferred_element_type=jnp.float32) + b1[...])
    out_ref[...] = _mlp_tail(h1, w2[...], b2[...], w3[...], b3[...],
                             gamma[...], beta[...]) + x


def _full(i):
    return (0, 0)


def _rows(i):
    return (i, 0)


_WSPEC = pl.BlockSpec((D, D), _full)
_VSPEC = pl.BlockSpec((1, D), _full)


def _edge_call(g, e, w1e, b1, w2, b2, w3, b3, gamma, beta):
    grid = (E // BE,)
    return pl.pallas_call(
        _edge_kernel,
        grid=grid,
        in_specs=[pl.BlockSpec((BE, D), _rows), pl.BlockSpec((BE, D), _rows),
                  _WSPEC, _VSPEC, _WSPEC, _VSPEC, _WSPEC, _VSPEC,
                  _VSPEC, _VSPEC],
        out_specs=pl.BlockSpec((BE, D), _rows),
        out_shape=jax.ShapeDtypeStruct((E, D), jnp.float32),
        compiler_params=pltpu.CompilerParams(
            dimension_semantics=("arbitrary",)),
    )(g, e, w1e, b1, w2, b2, w3, b3, gamma, beta)


def _pre_call(x, wd, ws):
    grid = (N // BN,)
    return pl.pallas_call(
        _pre_kernel,
        grid=grid,
        in_specs=[pl.BlockSpec((BN, D), _rows), _WSPEC, _WSPEC],
        out_specs=[pl.BlockSpec((BN, D), _rows), pl.BlockSpec((BN, D), _rows)],
        out_shape=[jax.ShapeDtypeStruct((N, D), jnp.float32),
                   jax.ShapeDtypeStruct((N, D), jnp.float32)],
        compiler_params=pltpu.CompilerParams(
            dimension_semantics=("arbitrary",)),
    )(x, wd, ws)


def _node_call(x, o0, o1, v1x, v1o, b1, w2, b2, w3, b3, gamma, beta):
    grid = (N // BN,)
    return pl.pallas_call(
        _node_kernel,
        grid=grid,
        in_specs=[pl.BlockSpec((BN, D), _rows), pl.BlockSpec((BN, D), _rows),
                  pl.BlockSpec((BN, D), _rows),
                  _WSPEC, _WSPEC, _VSPEC, _WSPEC, _VSPEC, _WSPEC, _VSPEC,
                  _VSPEC, _VSPEC],
        out_specs=pl.BlockSpec((BN, D), _rows),
        out_shape=jax.ShapeDtypeStruct((N, D), jnp.float32),
        compiler_params=pltpu.CompilerParams(
            dimension_semantics=("arbitrary",)),
    )(x, o0, o1, v1x, v1o, b1, w2, b2, w3, b3, gamma, beta)


def _row(v):
    return v.reshape(1, D)


def kernel(x, edge_index, edge_attr, params):
    dst3 = edge_index[1].reshape(NW, CH, K)
    src3 = edge_index[0].reshape(NW, CH, K)
    icd = edge_index[1].reshape(NW, CHG, KG)
    ics = edge_index[0].reshape(NW, CHG, KG)
    ic = jnp.stack([icd, ics], axis=2)
    e = edge_attr
    for p in params:
        em = p["edge_mlp"]
        nm = p["node_mlp"]
        w1, b1 = em["l1"]
        w2, b2 = em["l2"]
        w3, b3 = em["l3"]
        gamma, beta = em["ln"]
        a, b = _pre_call(x, w1[:D], w1[D:2 * D])
        a_pad = jnp.pad(a, ((0, NPAD - N), (0, 0)))
        g = _sc_gather(a_pad, b, ic)
        e_new = _edge_call(g, e, w1[2 * D:], _row(b1), w2, _row(b2),
                           w3, _row(b3), _row(gamma), _row(beta))
        parts = _sc_scatter(e_new, dst3)
        o0 = parts[0, :N]
        o1 = parts[1, :N]
        v1, c1 = nm["l1"]
        v2, c2 = nm["l2"]
        v3, c3 = nm["l3"]
        ngamma, nbeta = nm["ln"]
        x = _node_call(x, o0, o1, v1[:D], v1[D:], _row(c1), v2, _row(c2),
                       v3, _row(c3), _row(ngamma), _row(nbeta))
        e = e_new
    return (x, e)


# speedup vs baseline: 1.0653x; 1.0241x over previous
import functools

import jax
import jax.numpy as jnp
from jax import lax
from jax.experimental import pallas as pl
from jax.experimental.pallas import tpu as pltpu
from jax.experimental.pallas import tpu_sc as plsc

N = 10000
E = 320000
D = 128

NW = 32
EPW = E // NW
K = 80
CH = EPW // K
NPAD = 10240
RPT = NPAD // 16

BE = 640
BN = 2000

_mesh = plsc.VectorSubcoreMesh(core_axis_name="c", subcore_axis_name="s")


DW = D // 2
_MSK = -65536


_NBUF = 4


def _gather_body(a_hbm, b_hbm, dst_hbm, src_hbm, g_hbm,
                 idxd, idxs, va, vb, sga, sgb, ss):
    c = lax.axis_index("c")
    s = lax.axis_index("s")
    wid = s * 2 + c
    pltpu.sync_copy(dst_hbm.at[wid], idxd)
    pltpu.sync_copy(src_hbm.at[wid], idxs)

    def start_gather(j, b):
        pltpu.async_copy(a_hbm.at[idxd.at[j]], va[b], sga[b])
        pltpu.async_copy(b_hbm.at[idxs.at[j]], vb[b], sgb[b])

    start_gather(0, 0)
    start_gather(1, 1)

    def do_chunk(j, b):
        nb = (b + 2) % _NBUF

        @pl.when(j >= 2)
        def _():
            pltpu.make_async_copy(va[nb], g_hbm.at[pl.ds(0, K)], ss[nb]).wait()

        @pl.when(j + 2 < CH)
        def _():
            start_gather(j + 2, nb)

        pltpu.make_async_copy(a_hbm.at[idxd.at[j]], va[b], sga[b]).wait()
        pltpu.make_async_copy(b_hbm.at[idxs.at[j]], vb[b], sgb[b]).wait()

        def row(r, carry2):
            for cc in range(8):
                sl = pl.ds(cc * 16, 16)
                va[b][r, sl] = va[b][r, sl] + vb[b][r, sl]
            return carry2

        lax.fori_loop(0, K, row, 0, unroll=2)
        pltpu.async_copy(va[b], g_hbm.at[pl.ds(wid * EPW + j * K, K)], ss[b])

    def quad(j4, carry):
        for b in range(_NBUF):
            do_chunk(j4 * _NBUF + b, b)
        return carry

    lax.fori_loop(0, CH // _NBUF, quad, 0)
    for t in range(CH - CH % _NBUF, CH):
        do_chunk(t, t % _NBUF)
    for t in range(CH - 2, CH):
        b = t % _NBUF
        pltpu.make_async_copy(va[b], g_hbm.at[pl.ds(0, K)], ss[b]).wait()


@functools.partial(
    pl.kernel,
    out_type=jax.ShapeDtypeStruct((E, D), jnp.float32),
    mesh=_mesh,
    scratch_types=(
        [pltpu.VMEM((CH, K), jnp.int32)] * 2
        + [pltpu.VMEM((K, D), jnp.float32)] * (2 * _NBUF)
        + [pltpu.SemaphoreType.DMA] * (3 * _NBUF)
    ),
)
def _sc_gather(a_hbm, b_hbm, dst_hbm, src_hbm, g_hbm, idxd, idxs, *rest):
    va = rest[0:_NBUF]
    vb = rest[_NBUF:2 * _NBUF]
    sga = rest[2 * _NBUF:3 * _NBUF]
    sgb = rest[3 * _NBUF:4 * _NBUF]
    ss = rest[4 * _NBUF:5 * _NBUF]
    _gather_body(a_hbm, b_hbm, dst_hbm, src_hbm, g_hbm,
                 idxd, idxs, va, vb, sga, sgb, ss)


@functools.partial(
    pl.kernel,
    out_type=jax.ShapeDtypeStruct((2, NPAD, D), jnp.float32),
    mesh=_mesh,
    scratch_types=[
        pltpu.VMEM((CH, K), jnp.int32),
        pltpu.VMEM((K, D), jnp.float32),
        pltpu.VMEM((K, D), jnp.float32),
        pltpu.VMEM_SHARED((NPAD, D), jnp.float32),
        pltpu.SemaphoreType.DMA,
        pltpu.SemaphoreType.DMA,
        pltpu.SemaphoreType.DMA,
        pltpu.SemaphoreType.DMA,
    ],
)
def _sc_scatter(enew_hbm, dst_hbm, out_hbm, idx, rows0, rows1, acc,
                sr0, sr1, sw0, sw1):
    c = lax.axis_index("c")
    s = lax.axis_index("s")
    wid = s * 2 + c
    rows = (rows0, rows1)
    sr = (sr0, sr1)
    sw = (sw0, sw1)

    def zrow(r, carry):
        for cc in range(8):
            rows0[r, pl.ds(cc * 16, 16)] = jnp.zeros((16,), jnp.float32)
        return carry

    lax.fori_loop(0, K, zrow, 0)

    def zacc(t, carry):
        pltpu.sync_copy(rows0, acc.at[pl.ds(s * RPT + t * K, K)])
        return carry

    lax.fori_loop(0, RPT // K, zacc, 0)
    plsc.subcore_barrier()

    pltpu.sync_copy(dst_hbm.at[wid], idx)
    pltpu.async_copy(enew_hbm.at[pl.ds(wid * EPW, K)], rows0, sr0)

    def do_chunk(j, b):
        @pl.when(j >= 1)
        def _():
            pltpu.make_async_copy(rows[1 - b], acc.at[idx.at[j]],
                                  sw[1 - b]).wait()

        @pl.when(j + 1 < CH)
        def _():
            pltpu.async_copy(enew_hbm.at[pl.ds(wid * EPW + (j + 1) * K, K)],
                             rows[1 - b], sr[1 - b])

        pltpu.make_async_copy(enew_hbm.at[pl.ds(wid * EPW + j * K, K)],
                              rows[b], sr[b]).wait()
        pltpu.async_copy(rows[b], acc.at[idx.at[j]], sw[b], add=True)

    def pair(j2, carry):
        do_chunk(j2 * 2, 0)
        do_chunk(j2 * 2 + 1, 1)
        return carry

    lax.fori_loop(0, CH // 2, pair, 0)
    if CH % 2:
        do_chunk(CH - 1, (CH - 1) % 2)
    pltpu.make_async_copy(rows[(CH - 1) % 2], acc.at[idx.at[CH - 1]],
                          sw[(CH - 1) % 2]).wait()
    plsc.subcore_barrier()

    pltpu.sync_copy(acc.at[pl.ds(s * RPT, RPT)], out_hbm.at[c].at[pl.ds(s * RPT, RPT)])


def _silu(v):
    return v * jax.nn.sigmoid(v)


def _bdot(u, w):
    return jnp.dot(u.astype(jnp.bfloat16), w.astype(jnp.bfloat16),
                   preferred_element_type=jnp.float32)


def _mlp_tail(h1, w2, b2, w3, b3, gamma, beta):
    h1 = _silu(h1)
    h2 = _silu(_bdot(h1, w2) + b2)
    v = _bdot(h2, w3) + b3
    mu = jnp.mean(v, axis=-1, keepdims=True)
    vc = v - mu
    var = jnp.mean(vc * vc, axis=-1, keepdims=True)
    return vc * lax.rsqrt(var + 1e-5) * gamma + beta


def _edge_kernel(g_ref, e_ref, w1e, b1, w2, b2, w3, b3, gamma, beta, out_ref):
    e = e_ref[...]
    h1 = g_ref[...] + _bdot(e, w1e[...]) + b1[...]
    out_ref[...] = _mlp_tail(h1, w2[...], b2[...], w3[...], b3[...],
                             gamma[...], beta[...]) + e


def _pre_kernel(x_ref, wd, ws, a_ref, b_ref):
    x = x_ref[...]
    a_ref[...] = _bdot(x, wd[...])
    b_ref[...] = _bdot(x, ws[...])


def _node_kernel(x_ref, o0_ref, o1_ref, v1x, v1o, b1, w2, b2, w3, b3,
                 gamma, beta, wd, ws, out_ref, a_ref, b_ref):
    x = x_ref[...]
    o = o0_ref[...] + o1_ref[...]
    h1 = (jnp.dot(x, v1x[...], preferred_element_type=jnp.float32)
          + jnp.dot(o, v1o[...], preferred_element_type=jnp.float32) + b1[...])
    xn = _mlp_tail(h1, w2[...], b2[...], w3[...], b3[...],
                   gamma[...], beta[...]) + x
    out_ref[...] = xn
    a_ref[...] = _bdot(xn, wd[...])
    b_ref[...] = _bdot(xn, ws[...])


def _full(i):
    return (0, 0)


def _rows(i):
    return (i, 0)


_WSPEC = pl.BlockSpec((D, D), _full)
_VSPEC = pl.BlockSpec((1, D), _full)


def _edge_call(g, e, w1e, b1, w2, b2, w3, b3, gamma, beta):
    grid = (E // BE,)
    return pl.pallas_call(
        _edge_kernel,
        grid=grid,
        in_specs=[pl.BlockSpec((BE, D), _rows), pl.BlockSpec((BE, D), _rows),
                  _WSPEC, _VSPEC, _WSPEC, _VSPEC, _WSPEC, _VSPEC,
                  _VSPEC, _VSPEC],
        out_specs=pl.BlockSpec((BE, D), _rows),
        out_shape=jax.ShapeDtypeStruct((E, D), jnp.float32),
        compiler_params=pltpu.CompilerParams(
            dimension_semantics=("arbitrary",)),
    )(g, e, w1e, b1, w2, b2, w3, b3, gamma, beta)


def _pre_call(x, wd, ws):
    grid = (N // BN,)
    return pl.pallas_call(
        _pre_kernel,
        grid=grid,
        in_specs=[pl.BlockSpec((BN, D), _rows), _WSPEC, _WSPEC],
        out_specs=[pl.BlockSpec((BN, D), _rows), pl.BlockSpec((BN, D), _rows)],
        out_shape=[jax.ShapeDtypeStruct((N, D), jnp.float32),
                   jax.ShapeDtypeStruct((N, D), jnp.float32)],
        compiler_params=pltpu.CompilerParams(
            dimension_semantics=("arbitrary",)),
    )(x, wd, ws)


def _node_call(x, o0, o1, v1x, v1o, b1, w2, b2, w3, b3, gamma, beta, wd, ws):
    grid = (N // BN,)
    return pl.pallas_call(
        _node_kernel,
        grid=grid,
        in_specs=[pl.BlockSpec((BN, D), _rows), pl.BlockSpec((BN, D), _rows),
                  pl.BlockSpec((BN, D), _rows),
                  _WSPEC, _WSPEC, _VSPEC, _WSPEC, _VSPEC, _WSPEC, _VSPEC,
                  _VSPEC, _VSPEC, _WSPEC, _WSPEC],
        out_specs=[pl.BlockSpec((BN, D), _rows), pl.BlockSpec((BN, D), _rows),
                   pl.BlockSpec((BN, D), _rows)],
        out_shape=[jax.ShapeDtypeStruct((N, D), jnp.float32),
                   jax.ShapeDtypeStruct((N, D), jnp.float32),
                   jax.ShapeDtypeStruct((N, D), jnp.float32)],
        compiler_params=pltpu.CompilerParams(
            dimension_semantics=("arbitrary",)),
    )(x, o0, o1, v1x, v1o, b1, w2, b2, w3, b3, gamma, beta, wd, ws)


def _row(v):
    return v.reshape(1, D)


def kernel(x, edge_index, edge_attr, params):
    dst3 = edge_index[1].reshape(NW, CH, K)
    src3 = edge_index[0].reshape(NW, CH, K)
    e = edge_attr
    nl = len(params)
    w1_0 = params[0]["edge_mlp"]["l1"][0]
    a, b = _pre_call(x, w1_0[:D], w1_0[D:2 * D])
    for li, p in enumerate(params):
        em = p["edge_mlp"]
        nm = p["node_mlp"]
        w1, b1 = em["l1"]
        w2, b2 = em["l2"]
        w3, b3 = em["l3"]
        gamma, beta = em["ln"]
        g = _sc_gather(a, b, dst3, src3)
        e_new = _edge_call(g, e, w1[2 * D:], _row(b1), w2, _row(b2),
                           w3, _row(b3), _row(gamma), _row(beta))
        parts = _sc_scatter(e_new, dst3)
        o0 = parts[0, :N]
        o1 = parts[1, :N]
        v1, c1 = nm["l1"]
        v2, c2 = nm["l2"]
        v3, c3 = nm["l3"]
        ngamma, nbeta = nm["ln"]
        w1_next = params[li + 1]["edge_mlp"]["l1"][0] if li + 1 < nl else w1_0
        x, a, b = _node_call(x, o0, o1, v1[:D], v1[D:], _row(c1), v2,
                             _row(c2), v3, _row(c3), _row(ngamma),
                             _row(nbeta), w1_next[:D], w1_next[D:2 * D])
        e = e_new
    return (x, e)


# speedup vs baseline: 1.4608x; 1.3712x over previous
import functools

import jax
import jax.numpy as jnp
from jax import lax
from jax.experimental import pallas as pl
from jax.experimental.pallas import tpu as pltpu
from jax.experimental.pallas import tpu_sc as plsc

N = 10000
E = 320000
D = 128

NW = 32
EPW = E // NW
K = 80
CH = EPW // K
NPAD = 10240
RPT = NPAD // 16

BE = 3200
BN = 2000

_mesh = plsc.VectorSubcoreMesh(core_axis_name="c", subcore_axis_name="s")


DW = D // 2
_MSK = -65536


_NBUF = 4


def _gather_body(a_hbm, b_hbm, dst_hbm, src_hbm, g_hbm,
                 idxd, idxs, va, vb, sga, sgb, ss):
    c = lax.axis_index("c")
    s = lax.axis_index("s")
    wid = s * 2 + c
    pltpu.sync_copy(dst_hbm.at[wid], idxd)
    pltpu.sync_copy(src_hbm.at[wid], idxs)

    def start_gather(j, b):
        pltpu.async_copy(a_hbm.at[idxd.at[j]], va[b], sga[b])
        pltpu.async_copy(b_hbm.at[idxs.at[j]], vb[b], sgb[b])

    start_gather(0, 0)
    start_gather(1, 1)

    def do_chunk(j, b):
        nb = (b + 2) % _NBUF

        @pl.when(j >= 2)
        def _():
            pltpu.make_async_copy(va[nb], g_hbm.at[pl.ds(0, K)], ss[nb]).wait()

        @pl.when(j + 2 < CH)
        def _():
            start_gather(j + 2, nb)

        pltpu.make_async_copy(a_hbm.at[idxd.at[j]], va[b], sga[b]).wait()
        pltpu.make_async_copy(b_hbm.at[idxs.at[j]], vb[b], sgb[b]).wait()

        def row(r, carry2):
            for cc in range(8):
                sl = pl.ds(cc * 16, 16)
                va[b][r, sl] = va[b][r, sl] + vb[b][r, sl]
            return carry2

        lax.fori_loop(0, K, row, 0, unroll=2)
        pltpu.async_copy(va[b], g_hbm.at[pl.ds(wid * EPW + j * K, K)], ss[b])

    def quad(j4, carry):
        for b in range(_NBUF):
            do_chunk(j4 * _NBUF + b, b)
        return carry

    lax.fori_loop(0, CH // _NBUF, quad, 0)
    for t in range(CH - CH % _NBUF, CH):
        do_chunk(t, t % _NBUF)
    for t in range(CH - 2, CH):
        b = t % _NBUF
        pltpu.make_async_copy(va[b], g_hbm.at[pl.ds(0, K)], ss[b]).wait()


@functools.partial(
    pl.kernel,
    out_type=jax.ShapeDtypeStruct((E, D), jnp.float32),
    mesh=_mesh,
    scratch_types=(
        [pltpu.VMEM((CH, K), jnp.int32)] * 2
        + [pltpu.VMEM((K, D), jnp.float32)] * (2 * _NBUF)
        + [pltpu.SemaphoreType.DMA] * (3 * _NBUF)
    ),
)
def _sc_gather(a_hbm, b_hbm, dst_hbm, src_hbm, g_hbm, idxd, idxs, *rest):
    va = rest[0:_NBUF]
    vb = rest[_NBUF:2 * _NBUF]
    sga = rest[2 * _NBUF:3 * _NBUF]
    sgb = rest[3 * _NBUF:4 * _NBUF]
    ss = rest[4 * _NBUF:5 * _NBUF]
    _gather_body(a_hbm, b_hbm, dst_hbm, src_hbm, g_hbm,
                 idxd, idxs, va, vb, sga, sgb, ss)


@functools.partial(
    pl.kernel,
    out_type=jax.ShapeDtypeStruct((2, NPAD, D), jnp.float32),
    mesh=_mesh,
    scratch_types=[
        pltpu.VMEM((CH, K), jnp.int32),
        pltpu.VMEM((K, D), jnp.float32),
        pltpu.VMEM((K, D), jnp.float32),
        pltpu.VMEM_SHARED((NPAD, D), jnp.float32),
        pltpu.SemaphoreType.DMA,
        pltpu.SemaphoreType.DMA,
        pltpu.SemaphoreType.DMA,
        pltpu.SemaphoreType.DMA,
    ],
)
def _sc_scatter(enew_hbm, dst_hbm, out_hbm, idx, rows0, rows1, acc,
                sr0, sr1, sw0, sw1):
    c = lax.axis_index("c")
    s = lax.axis_index("s")
    wid = s * 2 + c
    rows = (rows0, rows1)
    sr = (sr0, sr1)
    sw = (sw0, sw1)

    def zrow(r, carry):
        for cc in range(8):
            rows0[r, pl.ds(cc * 16, 16)] = jnp.zeros((16,), jnp.float32)
        return carry

    lax.fori_loop(0, K, zrow, 0)

    def zacc(t, carry):
        pltpu.sync_copy(rows0, acc.at[pl.ds(s * RPT + t * K, K)])
        return carry

    lax.fori_loop(0, RPT // K, zacc, 0)
    plsc.subcore_barrier()

    pltpu.sync_copy(dst_hbm.at[wid], idx)
    pltpu.async_copy(enew_hbm.at[pl.ds(wid * EPW, K)], rows0, sr0)

    def do_chunk(j, b):
        @pl.when(j >= 1)
        def _():
            pltpu.make_async_copy(rows[1 - b], acc.at[idx.at[j]],
                                  sw[1 - b]).wait()

        @pl.when(j + 1 < CH)
        def _():
            pltpu.async_copy(enew_hbm.at[pl.ds(wid * EPW + (j + 1) * K, K)],
                             rows[1 - b], sr[1 - b])

        pltpu.make_async_copy(enew_hbm.at[pl.ds(wid * EPW + j * K, K)],
                              rows[b], sr[b]).wait()
        pltpu.async_copy(rows[b], acc.at[idx.at[j]], sw[b], add=True)

    def pair(j2, carry):
        do_chunk(j2 * 2, 0)
        do_chunk(j2 * 2 + 1, 1)
        return carry

    lax.fori_loop(0, CH // 2, pair, 0)
    if CH % 2:
        do_chunk(CH - 1, (CH - 1) % 2)
    pltpu.make_async_copy(rows[(CH - 1) % 2], acc.at[idx.at[CH - 1]],
                          sw[(CH - 1) % 2]).wait()
    plsc.subcore_barrier()

    pltpu.sync_copy(acc.at[pl.ds(s * RPT, RPT)], out_hbm.at[c].at[pl.ds(s * RPT, RPT)])


def _silu(v):
    return v * jax.nn.sigmoid(v)


def _bdot(u, w):
    return jnp.dot(u.astype(jnp.bfloat16), w.astype(jnp.bfloat16),
                   preferred_element_type=jnp.float32)


def _mlp_tail(h1, w2, b2, w3, b3, gamma, beta):
    h1 = _silu(h1)
    h2 = _silu(_bdot(h1, w2) + b2)
    v = _bdot(h2, w3) + b3
    mu = jnp.mean(v, axis=-1, keepdims=True)
    vc = v - mu
    var = jnp.mean(vc * vc, axis=-1, keepdims=True)
    return vc * lax.rsqrt(var + 1e-5) * gamma + beta


def _edge_kernel(g_ref, e_ref, w1e, b1, w2, b2, w3, b3, gamma, beta, out_ref):
    e = e_ref[...]
    h1 = g_ref[...] + _bdot(e, w1e[...]) + b1[...]
    out_ref[...] = _mlp_tail(h1, w2[...], b2[...], w3[...], b3[...],
                             gamma[...], beta[...]) + e


def _pre_kernel(x_ref, wd, ws, a_ref, b_ref):
    x = x_ref[...]
    a_ref[...] = _bdot(x, wd[...])
    b_ref[...] = _bdot(x, ws[...])


def _node_kernel(x_ref, o0_ref, o1_ref, v1x, v1o, b1, w2, b2, w3, b3,
                 gamma, beta, wd, ws, out_ref, a_ref, b_ref):
    x = x_ref[...]
    o = o0_ref[...] + o1_ref[...]
    h1 = (jnp.dot(x, v1x[...], preferred_element_type=jnp.float32)
          + jnp.dot(o, v1o[...], preferred_element_type=jnp.float32) + b1[...])
    xn = _mlp_tail(h1, w2[...], b2[...], w3[...], b3[...],
                   gamma[...], beta[...]) + x
    out_ref[...] = xn
    a_ref[...] = _bdot(xn, wd[...])
    b_ref[...] = _bdot(xn, ws[...])


def _full(i):
    return (0, 0)


def _rows(i):
    return (i, 0)


_WSPEC = pl.BlockSpec((D, D), _full)
_VSPEC = pl.BlockSpec((1, D), _full)


def _edge_call(g, e, w1e, b1, w2, b2, w3, b3, gamma, beta):
    grid = (E // BE,)
    return pl.pallas_call(
        _edge_kernel,
        grid=grid,
        in_specs=[pl.BlockSpec((BE, D), _rows), pl.BlockSpec((BE, D), _rows),
                  _WSPEC, _VSPEC, _WSPEC, _VSPEC, _WSPEC, _VSPEC,
                  _VSPEC, _VSPEC],
        out_specs=pl.BlockSpec((BE, D), _rows),
        out_shape=jax.ShapeDtypeStruct((E, D), jnp.float32),
        compiler_params=pltpu.CompilerParams(
            dimension_semantics=("arbitrary",)),
    )(g, e, w1e, b1, w2, b2, w3, b3, gamma, beta)


def _pre_call(x, wd, ws):
    grid = (N // BN,)
    return pl.pallas_call(
        _pre_kernel,
        grid=grid,
        in_specs=[pl.BlockSpec((BN, D), _rows), _WSPEC, _WSPEC],
        out_specs=[pl.BlockSpec((BN, D), _rows), pl.BlockSpec((BN, D), _rows)],
        out_shape=[jax.ShapeDtypeStruct((N, D), jnp.float32),
                   jax.ShapeDtypeStruct((N, D), jnp.float32)],
        compiler_params=pltpu.CompilerParams(
            dimension_semantics=("arbitrary",)),
    )(x, wd, ws)


def _node_call(x, o0, o1, v1x, v1o, b1, w2, b2, w3, b3, gamma, beta, wd, ws):
    grid = (N // BN,)
    return pl.pallas_call(
        _node_kernel,
        grid=grid,
        in_specs=[pl.BlockSpec((BN, D), _rows), pl.BlockSpec((BN, D), _rows),
                  pl.BlockSpec((BN, D), _rows),
                  _WSPEC, _WSPEC, _VSPEC, _WSPEC, _VSPEC, _WSPEC, _VSPEC,
                  _VSPEC, _VSPEC, _WSPEC, _WSPEC],
        out_specs=[pl.BlockSpec((BN, D), _rows), pl.BlockSpec((BN, D), _rows),
                   pl.BlockSpec((BN, D), _rows)],
        out_shape=[jax.ShapeDtypeStruct((N, D), jnp.float32),
                   jax.ShapeDtypeStruct((N, D), jnp.float32),
                   jax.ShapeDtypeStruct((N, D), jnp.float32)],
        compiler_params=pltpu.CompilerParams(
            dimension_semantics=("arbitrary",)),
    )(x, o0, o1, v1x, v1o, b1, w2, b2, w3, b3, gamma, beta, wd, ws)


def _row(v):
    return v.reshape(1, D)


def kernel(x, edge_index, edge_attr, params):
    dst3 = edge_index[1].reshape(NW, CH, K)
    src3 = edge_index[0].reshape(NW, CH, K)
    e = edge_attr
    nl = len(params)
    w1_0 = params[0]["edge_mlp"]["l1"][0]
    a, b = _pre_call(x, w1_0[:D], w1_0[D:2 * D])
    for li, p in enumerate(params):
        em = p["edge_mlp"]
        nm = p["node_mlp"]
        w1, b1 = em["l1"]
        w2, b2 = em["l2"]
        w3, b3 = em["l3"]
        gamma, beta = em["ln"]
        g = _sc_gather(a, b, dst3, src3)
        e_new = _edge_call(g, e, w1[2 * D:], _row(b1), w2, _row(b2),
                           w3, _row(b3), _row(gamma), _row(beta))
        parts = _sc_scatter(e_new, dst3)
        o0 = parts[0, :N]
        o1 = parts[1, :N]
        v1, c1 = nm["l1"]
        v2, c2 = nm["l2"]
        v3, c3 = nm["l3"]
        ngamma, nbeta = nm["ln"]
        w1_next = params[li + 1]["edge_mlp"]["l1"][0] if li + 1 < nl else w1_0
        x, a, b = _node_call(x, o0, o1, v1[:D], v1[D:], _row(c1), v2,
                             _row(c2), v3, _row(c3), _row(ngamma),
                             _row(nbeta), w1_next[:D], w1_next[D:2 * D])
        e = e_new
    return (x, e)
